# baseline (speedup 1.0000x reference)
import jax
import jax.numpy as jnp
from jax.experimental import pallas as pl
from jax.experimental.pallas import tpu as pltpu


def _fused_kernel(text_ref, adj_ref, Wn_ref, bn_ref, gamma_ref, beta_ref,
                  pn_ref, Wg_ref, bg_ref, pg_ref, out_ref, xl_ref):
    B, L, D = text_ref.shape
    i = pl.program_id(0)

    @pl.when(i == 0)
    def _mlp():
        x = text_ref[...].reshape(B * L, D)
        h = jnp.dot(x, Wn_ref[...], preferred_element_type=jnp.float32)
        h = h + bn_ref[...]
        mean = jnp.mean(h, axis=0, keepdims=True)
        var = jnp.mean((h - mean) * (h - mean), axis=0, keepdims=True)
        h = (h - mean) * jax.lax.rsqrt(var + 1e-5) * gamma_ref[...] + beta_ref[...]
        pn = pn_ref[0, 0]
        tn = jnp.where(h >= 0, h, pn * h)
        xl_ref[...] = jnp.dot(tn, Wg_ref[...], preferred_element_type=jnp.float32)

    row = jax.lax.broadcasted_iota(jnp.int32, (L, L), 0)
    col = jax.lax.broadcasted_iota(jnp.int32, (L, L), 1)
    A = jnp.where(row == col, 1.0, adj_ref[0].astype(jnp.float32))

    dn = (((0,), (0,)), ((), ()))
    ones_col = jnp.ones((L, 1), dtype=jnp.float32)
    deg = jax.lax.dot_general(A, ones_col, dn, preferred_element_type=jnp.float32)
    dinv = jax.lax.rsqrt(deg)
    msg = xl_ref[pl.ds(i * L, L), :] * dinv
    agg = jax.lax.dot_general(A, msg, dn, preferred_element_type=jnp.float32)
    hid = agg * dinv + bg_ref[...]
    pg = pg_ref[0, 0]
    g = jnp.where(hid >= 0, hid, pg * hid)
    nrm = jnp.sqrt(jnp.sum(g * g, axis=1, keepdims=True))
    g = g / jnp.maximum(nrm, 1e-12)
    out_ref[0] = g + text_ref[i]


def kernel(text_feature, adj, W_node, b_node, bn_gamma, bn_beta, prelu_node,
           W_gcn, b_gcn, prelu_gcn):
    B, L, D = text_feature.shape
    full = lambda shape: pl.BlockSpec(shape, lambda i: (0,) * len(shape))
    return pl.pallas_call(
        _fused_kernel,
        grid=(B,),
        in_specs=[
            full((B, L, D)),
            pl.BlockSpec((1, L, L), lambda i: (i, 0, 0)),
            full((D, D)),
            full((1, D)), full((1, D)), full((1, D)),
            full((1, 1)),
            full((D, D)),
            full((1, D)),
            full((1, 1)),
        ],
        out_specs=pl.BlockSpec((1, L, D), lambda i: (i, 0, 0)),
        out_shape=jax.ShapeDtypeStruct((B, L, D), jnp.float32),
        scratch_shapes=[pltpu.VMEM((B * L, D), jnp.float32)],
    )(text_feature, adj, W_node,
      b_node.reshape(1, D), bn_gamma.reshape(1, D), bn_beta.reshape(1, D),
      prelu_node.reshape(1, 1), W_gcn, b_gcn.reshape(1, D),
      prelu_gcn.reshape(1, 1))


# speedup vs baseline: 1.2502x; 1.2502x over previous
import jax
import jax.numpy as jnp
from jax.experimental import pallas as pl
from jax.experimental.pallas import tpu as pltpu


def _fused_kernel(text_ref, adj_ref, Wn_ref, bn_ref, gamma_ref, beta_ref,
                  pn_ref, Wg_ref, bg_ref, pg_ref, out_ref):
    B, L, D = text_ref.shape
    x = text_ref[...].reshape(B * L, D)

    h = jnp.dot(x, Wn_ref[...], preferred_element_type=jnp.float32) + bn_ref[...]
    mean = jnp.mean(h, axis=0, keepdims=True)
    var = jnp.mean((h - mean) * (h - mean), axis=0, keepdims=True)
    h = (h - mean) * jax.lax.rsqrt(var + 1e-5) * gamma_ref[...] + beta_ref[...]
    pn = pn_ref[0, 0]
    tn = jnp.where(h >= 0, h, pn * h)

    xl = jnp.dot(tn, Wg_ref[...], preferred_element_type=jnp.float32)

    pg = pg_ref[0, 0]
    ones_col = jnp.ones((L, 1), dtype=jnp.float32)
    row = jax.lax.broadcasted_iota(jnp.int32, (L, L), 0)
    col = jax.lax.broadcasted_iota(jnp.int32, (L, L), 1)
    diag = (row == col)

    dn = (((0,), (0,)), ((), ()))
    for b in range(B):
        A = jnp.where(diag, 1.0, adj_ref[b].astype(jnp.float32))
        deg = jax.lax.dot_general(A, ones_col, dn,
                                  preferred_element_type=jnp.float32)
        dinv = jax.lax.rsqrt(deg)
        msg = xl[b * L:(b + 1) * L] * dinv
        agg = jax.lax.dot_general(A, msg, dn,
                                  preferred_element_type=jnp.float32)
        hid = agg * dinv + bg_ref[...]
        g = jnp.where(hid >= 0, hid, pg * hid)
        nrm = jnp.sqrt(jnp.sum(g * g, axis=1, keepdims=True))
        g = g / jnp.maximum(nrm, 1e-12)
        out_ref[b] = g + text_ref[b]


def kernel(text_feature, adj, W_node, b_node, bn_gamma, bn_beta, prelu_node,
           W_gcn, b_gcn, prelu_gcn):
    B, L, D = text_feature.shape
    return pl.pallas_call(
        _fused_kernel,
        out_shape=jax.ShapeDtypeStruct((B, L, D), jnp.float32),
    )(text_feature, adj, W_node,
      b_node.reshape(1, D), bn_gamma.reshape(1, D), bn_beta.reshape(1, D),
      prelu_node.reshape(1, 1), W_gcn, b_gcn.reshape(1, D),
      prelu_gcn.reshape(1, 1))
